# Initial kernel scaffold; baseline (speedup 1.0000x reference)
import jax
import jax.numpy as jnp
from jax.experimental import pallas as pl


def kernel(x, token_mask, weights, indices, gate_projs, up_projs, down_projs):
    raise NotImplementedError("write your pallas kernel here")


# speedup vs baseline: 1.5480x; 1.5480x over previous
import functools

import jax
import jax.numpy as jnp
from jax import lax
from jax.experimental import pallas as pl
from jax.experimental.pallas import tpu as pltpu
from jax.experimental.pallas import tpu_sc as plsc

TM = 256
GATHER_CH = 48
COMBINE_CT = 8


def _sc_mesh():
    return plsc.VectorSubcoreMesh(core_axis_name="c", subcore_axis_name="s")


def _num_workers():
    info = plsc.get_sparse_core_info()
    return info.num_cores, info.num_subcores, info.num_cores * info.num_subcores


def _make_gather(P, D, nc, nw):
    rows_per_w = P // nw
    n_chunks = rows_per_w // GATHER_CH

    @functools.partial(
        pl.kernel,
        out_type=jax.ShapeDtypeStruct((P, D), jnp.float32),
        mesh=_sc_mesh(),
        scratch_types=[
            pltpu.VMEM((rows_per_w,), jnp.int32),
            pltpu.VMEM((GATHER_CH, D), jnp.float32),
            pltpu.SemaphoreType.DMA,
        ],
    )
    def gather_k(tok_hbm, x_hbm, xs_hbm, idx_v, rows_v, sem):
        wid = lax.axis_index("s") * nc + lax.axis_index("c")
        base = wid * rows_per_w
        pltpu.sync_copy(tok_hbm.at[pl.ds(base, rows_per_w)], idx_v)

        def chunk(ci, carry):
            off = ci * GATHER_CH
            pltpu.async_copy(
                x_hbm.at[idx_v.at[pl.ds(off, GATHER_CH)]], rows_v, sem
            ).wait()
            pltpu.sync_copy(rows_v, xs_hbm.at[pl.ds(base + off, GATHER_CH)])
            return carry

        lax.fori_loop(0, n_chunks, chunk, 0)

    return gather_k


def _make_combine(T, D, P, K, nc, nw):
    toks_per_w = T // nw
    n_chunks = toks_per_w // COMBINE_CT

    @functools.partial(
        pl.kernel,
        out_type=jax.ShapeDtypeStruct((T, D), jnp.float32),
        mesh=_sc_mesh(),
        scratch_types=[
            pltpu.VMEM((toks_per_w,), jnp.int32),
            pltpu.VMEM((toks_per_w,), jnp.int32),
            pltpu.VMEM((COMBINE_CT, D), jnp.float32),
            pltpu.VMEM((COMBINE_CT, D), jnp.float32),
            pltpu.SemaphoreType.DMA,
            pltpu.SemaphoreType.DMA,
        ],
    )
    def combine_k(pa_hbm, pb_hbm, ys_hbm, y_hbm, ia_v, ib_v, ra_v, rb_v, sa, sb):
        wid = lax.axis_index("s") * nc + lax.axis_index("c")
        base = wid * toks_per_w
        pltpu.sync_copy(pa_hbm.at[pl.ds(base, toks_per_w)], ia_v)
        pltpu.sync_copy(pb_hbm.at[pl.ds(base, toks_per_w)], ib_v)

        def chunk(ci, carry):
            off = ci * COMBINE_CT
            cpa = pltpu.async_copy(
                ys_hbm.at[ia_v.at[pl.ds(off, COMBINE_CT)]], ra_v, sa)
            cpb = pltpu.async_copy(
                ys_hbm.at[ib_v.at[pl.ds(off, COMBINE_CT)]], rb_v, sb)
            cpa.wait()
            cpb.wait()

            def rowadd(r, c2):
                for cc in range(D // 16):
                    sl = pl.ds(cc * 16, 16)
                    ra_v[r, sl] = ra_v[r, sl] + rb_v[r, sl]
                return c2

            lax.fori_loop(0, COMBINE_CT, rowadd, 0)
            pltpu.sync_copy(ra_v, y_hbm.at[pl.ds(base + off, COMBINE_CT)])
            return carry

        lax.fori_loop(0, n_chunks, chunk, 0)

    return combine_k


def _tc_swiglu_body(te_ref, xs_ref, rw_ref, g_ref, u_ref, d_ref, o_ref):
    xt = xs_ref[...].astype(jnp.bfloat16)
    g = g_ref[0]
    u = u_ref[0]
    dn = d_ref[0]
    a = lax.dot_general(xt, g, (((1,), (1,)), ((), ())),
                        preferred_element_type=jnp.float32)
    b = lax.dot_general(xt, u, (((1,), (1,)), ((), ())),
                        preferred_element_type=jnp.float32)
    h = ((a * jax.nn.sigmoid(a)) * b).astype(jnp.bfloat16)
    o = lax.dot_general(h, dn, (((1,), (1,)), ((), ())),
                        preferred_element_type=jnp.float32)
    o_ref[...] = o * rw_ref[...]


def _make_grouped_swiglu(P, D, FF, E, ntiles):
    grid_spec = pltpu.PrefetchScalarGridSpec(
        num_scalar_prefetch=1,
        grid=(ntiles,),
        in_specs=[
            pl.BlockSpec((TM, D), lambda i, te: (i, 0)),
            pl.BlockSpec((TM, 1), lambda i, te: (i, 0)),
            pl.BlockSpec((1, FF, D), lambda i, te: (te[i], 0, 0)),
            pl.BlockSpec((1, FF, D), lambda i, te: (te[i], 0, 0)),
            pl.BlockSpec((1, D, FF), lambda i, te: (te[i], 0, 0)),
        ],
        out_specs=pl.BlockSpec((TM, D), lambda i, te: (i, 0)),
    )
    return pl.pallas_call(
        _tc_swiglu_body,
        grid_spec=grid_spec,
        out_shape=jax.ShapeDtypeStruct((P, D), jnp.float32),
        compiler_params=pltpu.CompilerParams(
            dimension_semantics=("arbitrary",),
        ),
    )


def kernel(x, token_mask, weights, indices, gate_projs, up_projs, down_projs):
    T, D = x.shape
    E, FF, _ = gate_projs.shape
    K = indices.shape[1]
    TK = T * K
    P = TK + E * TM
    ntiles = P // TM
    nc, _, nw = _num_workers()

    e_flat = indices.reshape(-1).astype(jnp.int32)
    w_flat = (weights * token_mask[:, None].astype(weights.dtype)).reshape(-1)
    oh = (e_flat[:, None] == jnp.arange(E, dtype=jnp.int32)[None, :]).astype(jnp.int32)
    cum = jnp.cumsum(oh, axis=0)
    counts = cum[-1]
    rank = jnp.take_along_axis(cum, e_flat[:, None], axis=1)[:, 0] - 1
    pcounts = ((counts + TM - 1) // TM) * TM
    poff = jnp.concatenate(
        [jnp.zeros((1,), jnp.int32), jnp.cumsum(pcounts)[:-1].astype(jnp.int32)])
    ppos = poff[e_flat] + rank
    tok = jnp.arange(TK, dtype=jnp.int32) // K
    row_token = jnp.zeros((P,), jnp.int32).at[ppos].set(tok)
    row_weight = jnp.zeros((P,), jnp.float32).at[ppos].set(w_flat)
    tile_expert = jnp.clip(
        jnp.searchsorted(poff, jnp.arange(ntiles, dtype=jnp.int32) * TM,
                         side="right") - 1,
        0, E - 1).astype(jnp.int32)
    pos = ppos.reshape(T, K)

    xs = _make_gather(P, D, nc, nw)(row_token, x)

    ys = _make_grouped_swiglu(P, D, FF, E, ntiles)(
        tile_expert, xs, row_weight.reshape(P, 1),
        gate_projs.astype(jnp.bfloat16), up_projs.astype(jnp.bfloat16),
        down_projs.astype(jnp.bfloat16))

    y = _make_combine(T, D, P, K, nc, nw)(
        pos[:, 0].astype(jnp.int32), pos[:, 1].astype(jnp.int32), ys)
    return y
